# Initial kernel scaffold; baseline (speedup 1.0000x reference)
import jax
import jax.numpy as jnp
from jax.experimental import pallas as pl


def kernel(x, edge_index, W_in, b_in, W_mid, b_mid, W_out, b_out, Wc, bc):
    raise NotImplementedError("write your pallas kernel here")


# speedup vs baseline: 7.4411x; 7.4411x over previous
import functools

import jax
import jax.numpy as jnp
from jax import lax
from jax.experimental import pallas as pl
from jax.experimental.pallas import tpu as pltpu
from jax.experimental.pallas import tpu_sc as plsc

N = 10000
D = 128
NPAD = 10240
CHUNK = 128
NC = 2
NS = 16
ROWS_PER_TILE = NPAD // NS
ROW_CHUNKS = ROWS_PER_TILE // CHUNK
MBLK = 1024
NBLK = NPAD // MBLK


def _sc_prop(hs_pad, src_pad, dst_pad, cpw):
    mesh = plsc.VectorSubcoreMesh(core_axis_name="c", subcore_axis_name="s")

    @functools.partial(
        pl.kernel,
        out_type=jax.ShapeDtypeStruct((NC, NPAD, D), jnp.float32),
        mesh=mesh,
        scratch_types=[
            pltpu.VMEM_SHARED((NPAD, D), jnp.float32),
            pltpu.VMEM((CHUNK,), jnp.int32),
            pltpu.VMEM((CHUNK,), jnp.int32),
            pltpu.VMEM((CHUNK, D), jnp.float32),
            pltpu.VMEM((CHUNK, D), jnp.float32),
            pltpu.SemaphoreType.DMA,
        ],
    )
    def kern(hs_hbm, src_hbm, dst_hbm, out_hbm, acc_sh, idx_s, idx_d, rows,
             zbuf, sem):
        c = lax.axis_index("c")
        s = lax.axis_index("s")

        zeros16 = jnp.zeros((16,), jnp.float32)

        def zrow(r, _):
            for j in range(D // 16):
                zbuf[r, pl.ds(j * 16, 16)] = zeros16
            return 0

        lax.fori_loop(0, CHUNK, zrow, 0)
        row0 = s * ROWS_PER_TILE
        for j in range(ROW_CHUNKS):
            pltpu.sync_copy(zbuf, acc_sh.at[pl.ds(row0 + j * CHUNK, CHUNK)])
        plsc.subcore_barrier()

        w = c * NS + s
        base0 = w * cpw * CHUNK

        def body(k, _):
            base = pl.multiple_of(base0 + k * CHUNK, CHUNK)
            pltpu.sync_copy(src_hbm.at[pl.ds(base, CHUNK)], idx_s)
            pltpu.sync_copy(dst_hbm.at[pl.ds(base, CHUNK)], idx_d)
            pltpu.async_copy(hs_hbm.at[idx_s], rows, sem).wait()
            pltpu.sync_copy(rows, acc_sh.at[idx_d], add=True)
            return 0

        lax.fori_loop(0, cpw, body, 0)
        plsc.subcore_barrier()

        for j in range(ROW_CHUNKS):
            r = row0 + j * CHUNK
            pltpu.sync_copy(acc_sh.at[pl.ds(r, CHUNK)], rows)
            pltpu.sync_copy(rows, out_hbm.at[c, pl.ds(r, CHUNK)])

    return kern(hs_pad, src_pad, dst_pad)


def _tc_first(deg_parts, x_pad, W_in):
    def kern(dp_ref, x_ref, w_ref, dis_ref, hs_ref):
        deg = dp_ref[0] + dp_ref[1] + 1.0
        dis = lax.rsqrt(deg)
        dis_ref[...] = dis
        h = jnp.dot(x_ref[...], w_ref[...], preferred_element_type=jnp.float32)
        hs_ref[...] = dis * h

    return pl.pallas_call(
        kern,
        grid=(NBLK,),
        in_specs=[
            pl.BlockSpec((NC, MBLK, D), lambda i: (0, i, 0)),
            pl.BlockSpec((MBLK, D), lambda i: (i, 0)),
            pl.BlockSpec((D, D), lambda i: (0, 0)),
        ],
        out_specs=[
            pl.BlockSpec((MBLK, D), lambda i: (i, 0)),
            pl.BlockSpec((MBLK, D), lambda i: (i, 0)),
        ],
        out_shape=[
            jax.ShapeDtypeStruct((NPAD, D), jnp.float32),
            jax.ShapeDtypeStruct((NPAD, D), jnp.float32),
        ],
    )(deg_parts, x_pad, W_in)


def _tc_mid(acc, hs_prev, dis_rep, b_row, W_next):
    def kern(a_ref, hp_ref, dis_ref, b_ref, w_ref, hs_ref):
        i = pl.program_id(0)
        dis = dis_ref[...]
        pre = dis * (a_ref[0] + a_ref[1] + hp_ref[...]) + b_ref[...]
        gid = i * MBLK + lax.broadcasted_iota(jnp.int32, (MBLK, D), 0)
        xn = jnp.where(gid < N, jnp.maximum(pre, 0.0), 0.0)
        h = jnp.dot(xn, w_ref[...], preferred_element_type=jnp.float32)
        hs_ref[...] = dis * h

    return pl.pallas_call(
        kern,
        grid=(NBLK,),
        in_specs=[
            pl.BlockSpec((NC, MBLK, D), lambda i: (0, i, 0)),
            pl.BlockSpec((MBLK, D), lambda i: (i, 0)),
            pl.BlockSpec((MBLK, D), lambda i: (i, 0)),
            pl.BlockSpec((1, D), lambda i: (0, 0)),
            pl.BlockSpec((D, D), lambda i: (0, 0)),
        ],
        out_specs=pl.BlockSpec((MBLK, D), lambda i: (i, 0)),
        out_shape=jax.ShapeDtypeStruct((NPAD, D), jnp.float32),
    )(acc, hs_prev, dis_rep, b_row, W_next)


def _tc_final(acc, hs3, dis_rep, b_row, Wc_pad, bc_row):
    def kern(a_ref, hp_ref, dis_ref, b_ref, wc_ref, bc_ref,
             ne_ref, ge_ref, lg_ref, ssum):
        i = pl.program_id(0)
        pre = dis_ref[...] * (a_ref[0] + a_ref[1] + hp_ref[...]) + b_ref[...]
        gid = i * MBLK + lax.broadcasted_iota(jnp.int32, (MBLK, D), 0)
        ne = jnp.where(gid < N, pre, 0.0)
        ne_ref[...] = ne
        csum = jnp.sum(ne, axis=0, keepdims=True)

        @pl.when(i == 0)
        def _():
            ssum[...] = csum

        @pl.when(i > 0)
        def _():
            ssum[...] = ssum[...] + csum

        @pl.when(i == NBLK - 1)
        def _():
            ge = ssum[...] * (1.0 / N)
            ge_ref[...] = ge
            lg_ref[...] = jnp.dot(ge, wc_ref[...],
                                  preferred_element_type=jnp.float32) + bc_ref[...]

    return pl.pallas_call(
        kern,
        grid=(NBLK,),
        in_specs=[
            pl.BlockSpec((NC, MBLK, D), lambda i: (0, i, 0)),
            pl.BlockSpec((MBLK, D), lambda i: (i, 0)),
            pl.BlockSpec((MBLK, D), lambda i: (i, 0)),
            pl.BlockSpec((1, D), lambda i: (0, 0)),
            pl.BlockSpec((D, D), lambda i: (0, 0)),
            pl.BlockSpec((1, D), lambda i: (0, 0)),
        ],
        out_specs=[
            pl.BlockSpec((MBLK, D), lambda i: (i, 0)),
            pl.BlockSpec((1, D), lambda i: (0, 0)),
            pl.BlockSpec((1, D), lambda i: (0, 0)),
        ],
        out_shape=[
            jax.ShapeDtypeStruct((NPAD, D), jnp.float32),
            jax.ShapeDtypeStruct((1, D), jnp.float32),
            jax.ShapeDtypeStruct((1, D), jnp.float32),
        ],
        scratch_shapes=[pltpu.VMEM((1, D), jnp.float32)],
    )(acc, hs3, dis_rep, b_row, Wc_pad, bc_row)


def kernel(x, edge_index, W_in, b_in, W_mid, b_mid, W_out, b_out, Wc, bc):
    E = edge_index.shape[1]
    cpw = -(-E // (NC * NS * CHUNK))
    e_pad = NC * NS * cpw * CHUNK

    x_pad = jnp.zeros((NPAD, D), jnp.float32).at[:N].set(x)
    sent = jnp.full((e_pad - E,), NPAD - 1, jnp.int32)
    src_pad = jnp.concatenate([edge_index[0], sent])
    dst_pad = jnp.concatenate([edge_index[1], sent])
    ones_pad = jnp.ones((NPAD, D), jnp.float32)
    Wc_pad = jnp.zeros((D, D), jnp.float32).at[:, :2].set(Wc)
    bc_row = jnp.zeros((1, D), jnp.float32).at[0, :2].set(bc)

    deg_parts = _sc_prop(ones_pad, src_pad, dst_pad, cpw)
    dis_rep, hs1 = _tc_first(deg_parts, x_pad, W_in)
    acc1 = _sc_prop(hs1, src_pad, dst_pad, cpw)
    hs2 = _tc_mid(acc1, hs1, dis_rep, b_in.reshape(1, D), W_mid)
    acc2 = _sc_prop(hs2, src_pad, dst_pad, cpw)
    hs3 = _tc_mid(acc2, hs2, dis_rep, b_mid.reshape(1, D), W_out)
    acc3 = _sc_prop(hs3, src_pad, dst_pad, cpw)
    ne_pad, ge_row, lg_row = _tc_final(acc3, hs3, dis_rep,
                                       b_out.reshape(1, D), Wc_pad, bc_row)

    node_embeddings = ne_pad[:N]
    graph_embedding = ge_row[0]
    logits = lg_row[0, :2]
    return (node_embeddings, graph_embedding, logits)
